# Initial kernel scaffold; baseline (speedup 1.0000x reference)
import jax
import jax.numpy as jnp
from jax.experimental import pallas as pl


def kernel(x, edge_index_low, edge_index_high, idx, W_self1, W_nbr1, b1, W_self2, W_nbr2, b2, W_self_skip, W_nbr_skip, b_skip):
    raise NotImplementedError("write your pallas kernel here")


# speedup vs baseline: 3.4726x; 3.4726x over previous
import jax
import jax.numpy as jnp
from jax import lax
from jax.experimental import pallas as pl
from jax.experimental.pallas import tpu as pltpu
from jax.experimental.pallas import tpu_sc as plsc

N_LOW = 5000
N_HIGH = 10000
C_IN = 128
C_MID = 64
C_OUT = 128

CP = 5120
FP = 10240
ELP = 163840
EHP = 327680
LOW_PAD_NODE = 5000
HIGH_PAD_NODE = 10008
IDX_PAD = 10000

_HIGHEST = jax.lax.Precision.HIGHEST


def _lrelu(v):
    return jnp.where(v >= 0, v, 0.01 * v)


def _dot(a, b):
    return jnp.dot(a, b, precision=_HIGHEST, preferred_element_type=jnp.float32)


def _tc1_body(x_ref, w_ref, y1_ref, s1_ref):
    h = _dot(x_ref[...], w_ref[...])
    y1_ref[...] = jnp.concatenate(
        [h[:, :C_MID], jnp.zeros_like(h[:, :C_MID])], axis=1)
    s1_ref[...] = h[:, C_MID:]


def _tc1(x_pad, w_cat):
    return pl.pallas_call(
        _tc1_body,
        grid=(8,),
        in_specs=[
            pl.BlockSpec((CP // 8, C_IN), lambda i: (i, 0)),
            pl.BlockSpec((C_IN, 2 * C_MID), lambda i: (0, 0)),
        ],
        out_specs=[
            pl.BlockSpec((CP // 8, 128), lambda i: (i, 0)),
            pl.BlockSpec((CP // 8, C_MID), lambda i: (i, 0)),
        ],
        out_shape=[
            jax.ShapeDtypeStruct((CP, 128), jnp.float32),
            jax.ShapeDtypeStruct((CP, C_MID), jnp.float32),
        ],
    )(x_pad, w_cat)


def _tc2_body(x_ref, s1_ref, agg_ref, b1_ref, m_ref):
    i = pl.program_id(0)
    blk = x_ref.shape[0]
    h1 = _lrelu(s1_ref[...] + agg_ref[0, :, :C_MID] + agg_ref[1, :, :C_MID]
                + b1_ref[...])
    row = jax.lax.broadcasted_iota(jnp.int32, (blk, 1), 0) + i * blk
    h1 = jnp.where(row < N_LOW, h1, 0.0)
    m_ref[0] = x_ref[...]
    m_ref[1] = jnp.concatenate([h1, jnp.zeros_like(h1)], axis=1)


def _tc2(x_pad, s1, agg_l, b1):
    blk = CP // 8
    return pl.pallas_call(
        _tc2_body,
        grid=(8,),
        in_specs=[
            pl.BlockSpec((blk, C_IN), lambda i: (i, 0)),
            pl.BlockSpec((blk, C_MID), lambda i: (i, 0)),
            pl.BlockSpec((2, blk, 128), lambda i: (0, i, 0)),
            pl.BlockSpec((1, C_MID), lambda i: (0, 0)),
        ],
        out_specs=pl.BlockSpec((2, blk, 128), lambda i: (0, i, 0)),
        out_shape=jax.ShapeDtypeStruct((2, CP, 128), jnp.float32),
    )(x_pad, s1, agg_l, b1)


def _tc3_body(u_ref, a_ref, wss, ws2, wns, wn2, b2_ref, bsk_ref, o_ref):
    xu, hu = u_ref[0], u_ref[1][:, :C_MID]
    ask, a2 = a_ref[0], a_ref[1][:, :C_MID]
    skip_pre = _dot(xu, wss[...]) + _dot(ask, wns[...]) + bsk_ref[...]
    h2_pre = _dot(hu, ws2[...]) + _dot(a2, wn2[...]) + b2_ref[...]
    o_ref[...] = _lrelu(_lrelu(h2_pre) + _lrelu(skip_pre))


def _tc3(u, agg, w_self_skip, w_self2, w_nbr_skip, w_nbr2, b2, b_skip):
    blk = FP // 16
    wmat = lambda m, n: pl.BlockSpec((m, n), lambda i: (0, 0))
    return pl.pallas_call(
        _tc3_body,
        grid=(16,),
        in_specs=[
            pl.BlockSpec((2, blk, 128), lambda i: (0, i, 0)),
            pl.BlockSpec((2, blk, 128), lambda i: (0, i, 0)),
            wmat(C_IN, C_OUT), wmat(C_MID, C_OUT),
            wmat(C_IN, C_OUT), wmat(C_MID, C_OUT),
            wmat(1, C_OUT), wmat(1, C_OUT),
        ],
        out_specs=pl.BlockSpec((blk, C_OUT), lambda i: (i, 0)),
        out_shape=jax.ShapeDtypeStruct((FP, C_OUT), jnp.float32),
    )(u, agg, w_self_skip, w_self2, w_nbr_skip, w_nbr2,
      b2.reshape(1, C_OUT), b_skip.reshape(1, C_OUT))


_SC_MESH = plsc.VectorSubcoreMesh(core_axis_name="c", subcore_axis_name="s")


def _sc_low_body(y1_hbm, src_hbm, dst_hbm, zeros_hbm, out_hbm,
                 acc_sh, src_v, dst_v, rows_v, buf_v):
    c = lax.axis_index("c")
    s = lax.axis_index("s")
    rows_per = CP // 16
    r0 = s * rows_per
    pltpu.sync_copy(zeros_hbm.at[pl.ds(0, 64)], buf_v)

    @pl.loop(0, 5)
    def _(k):
        pltpu.sync_copy(buf_v, acc_sh.at[pl.ds(r0 + k * 64, 64)])

    plsc.subcore_barrier()
    w = c * 16 + s

    @pl.loop(0, 5)
    def _(g):
        pltpu.sync_copy(src_hbm.at[pl.ds(w * 40 + g * 8, 8)], src_v)
        pltpu.sync_copy(dst_hbm.at[pl.ds(w * 40 + g * 8, 8)], dst_v)

        @pl.loop(0, 8)
        def _(j):
            pltpu.sync_copy(y1_hbm.at[src_v.at[j]], rows_v)
            pltpu.sync_copy(rows_v, acc_sh.at[dst_v.at[j]], add=True)

    plsc.subcore_barrier()

    @pl.loop(0, 5)
    def _(k):
        pltpu.sync_copy(acc_sh.at[pl.ds(r0 + k * 64, 64)], buf_v)
        pltpu.sync_copy(buf_v, out_hbm.at[c, pl.ds(r0 + k * 64, 64)])


def _sc_low(y1_pad, src_r, dst_r, zeros_l):
    f = pl.kernel(
        _sc_low_body,
        out_type=jax.ShapeDtypeStruct((2, CP, 128), jnp.float32),
        mesh=_SC_MESH,
        scratch_types=[
            pltpu.VMEM_SHARED((CP, 128), jnp.float32),
            pltpu.VMEM((8, 128), jnp.int32),
            pltpu.VMEM((8, 128), jnp.int32),
            pltpu.VMEM((128, 128), jnp.float32),
            pltpu.VMEM((64, 128), jnp.float32),
        ],
    )
    return f(y1_pad, src_r, dst_r, zeros_l)


def _sc_high_body(m_hbm, idx_hbm, src_hbm, dst_hbm, zeros_hbm,
                  agg_hbm, u_hbm,
                  acc_sh, sidx_v, buf_v, src_v, dst_v, rows_v):
    c = lax.axis_index("c")
    s = lax.axis_index("s")
    rows_per = FP // 16
    r0 = s * rows_per
    pltpu.sync_copy(zeros_hbm.at[pl.ds(0, 64)], buf_v)

    @pl.loop(0, 10)
    def _(k):
        pltpu.sync_copy(buf_v, acc_sh.at[pl.ds(r0 + k * 64, 64)])
        pltpu.sync_copy(buf_v, u_hbm.at[c, pl.ds(r0 + k * 64, 64)])

    plsc.subcore_barrier()
    pltpu.sync_copy(idx_hbm.at[s], sidx_v)

    @pl.loop(0, 5)
    def _(j):
        pltpu.sync_copy(m_hbm.at[c, pl.ds(s * 320 + j * 64, 64)], buf_v)
        pltpu.sync_copy(buf_v, u_hbm.at[c].at[sidx_v.at[j]])

    plsc.subcore_barrier()
    @pl.loop(0, 10)
    def _(g):
        pltpu.sync_copy(src_hbm.at[pl.ds(s * 160 + g * 16, 16)], src_v)
        pltpu.sync_copy(dst_hbm.at[pl.ds(s * 160 + g * 16, 16)], dst_v)

        @pl.loop(0, 16)
        def _(j):
            pltpu.sync_copy(u_hbm.at[c].at[src_v.at[j]], rows_v)
            pltpu.sync_copy(rows_v, acc_sh.at[dst_v.at[j]], add=True)

    plsc.subcore_barrier()

    @pl.loop(0, 10)
    def _(k):
        pltpu.sync_copy(acc_sh.at[pl.ds(r0 + k * 64, 64)], buf_v)
        pltpu.sync_copy(buf_v, agg_hbm.at[c, pl.ds(r0 + k * 64, 64)])


def _sc_high(m, idx_r, src_r, dst_r, zeros_h):
    f = pl.kernel(
        _sc_high_body,
        out_type=[
            jax.ShapeDtypeStruct((2, FP, 128), jnp.float32),
            jax.ShapeDtypeStruct((2, FP, 128), jnp.float32),
        ],
        mesh=_SC_MESH,
        scratch_types=[
            pltpu.VMEM_SHARED((FP, 128), jnp.float32),
            pltpu.VMEM((8, 64), jnp.int32),
            pltpu.VMEM((64, 128), jnp.float32),
            pltpu.VMEM((16, 128), jnp.int32),
            pltpu.VMEM((16, 128), jnp.int32),
            pltpu.VMEM((128, 128), jnp.float32),
        ],
    )
    return f(m, idx_r, src_r, dst_r, zeros_h)


def _pad_edges(e, total, pad_val):
    pad = jnp.full((total - e.shape[0],), pad_val, dtype=jnp.int32)
    return jnp.concatenate([e, pad]).reshape(total // 128, 128)


def kernel(x, edge_index_low, edge_index_high, idx,
           W_self1, W_nbr1, b1, W_self2, W_nbr2, b2,
           W_self_skip, W_nbr_skip, b_skip):
    x_pad = jnp.concatenate(
        [x, jnp.zeros((CP - N_LOW, C_IN), jnp.float32)], axis=0)
    w_cat = jnp.concatenate([W_nbr1, W_self1], axis=1)

    src_l = _pad_edges(edge_index_low[0], ELP, LOW_PAD_NODE)
    dst_l = _pad_edges(edge_index_low[1], ELP, LOW_PAD_NODE)
    src_h = _pad_edges(edge_index_high[0], EHP, HIGH_PAD_NODE)
    dst_h = _pad_edges(edge_index_high[1], EHP, HIGH_PAD_NODE)
    idx_r = jnp.concatenate(
        [idx, jnp.full((CP - N_LOW,), IDX_PAD, jnp.int32)]).reshape(16, 5, 64)
    idx_r = jnp.concatenate(
        [idx_r, jnp.full((16, 3, 64), IDX_PAD, jnp.int32)], axis=1)

    zeros_l = jnp.zeros((CP, 128), jnp.float32)
    zeros_h = jnp.zeros((FP, 128), jnp.float32)

    y1, s1 = _tc1(x_pad, w_cat)
    agg_l = _sc_low(y1, src_l, dst_l, zeros_l)
    m = _tc2(x_pad, s1, agg_l, b1.reshape(1, C_MID))
    agg_h, u = _sc_high(m, idx_r, src_h, dst_h, zeros_h)
    out = _tc3(u, agg_h, W_self_skip, W_self2, W_nbr_skip, W_nbr2,
               b2, b_skip)
    return out[:N_HIGH]


# speedup vs baseline: 3.6611x; 1.0543x over previous
import jax
import jax.numpy as jnp
from jax import lax
from jax.experimental import pallas as pl
from jax.experimental.pallas import tpu as pltpu
from jax.experimental.pallas import tpu_sc as plsc

N_LOW = 5000
N_HIGH = 10000
C_IN = 128
C_MID = 64
C_OUT = 128

CP = 5120
FP = 10240
ELP = 163840
EHP = 327680
LOW_PAD_NODE = 5000
HIGH_PAD_NODE = 10008
IDX_PAD = 10000

_HIGHEST = jax.lax.Precision.HIGHEST


def _lrelu(v):
    return jnp.where(v >= 0, v, 0.01 * v)


def _dot(a, b):
    return jnp.dot(a, b, precision=_HIGHEST, preferred_element_type=jnp.float32)


def _tc1_body(x_ref, w_ref, y1_ref, s1_ref):
    h = _dot(x_ref[...], w_ref[...])
    y1_ref[...] = jnp.concatenate(
        [h[:, :C_MID], jnp.zeros_like(h[:, :C_MID])], axis=1)
    s1_ref[...] = h[:, C_MID:]


def _tc1(x_pad, w_cat):
    return pl.pallas_call(
        _tc1_body,
        grid=(8,),
        in_specs=[
            pl.BlockSpec((CP // 8, C_IN), lambda i: (i, 0)),
            pl.BlockSpec((C_IN, 2 * C_MID), lambda i: (0, 0)),
        ],
        out_specs=[
            pl.BlockSpec((CP // 8, 128), lambda i: (i, 0)),
            pl.BlockSpec((CP // 8, C_MID), lambda i: (i, 0)),
        ],
        out_shape=[
            jax.ShapeDtypeStruct((CP, 128), jnp.float32),
            jax.ShapeDtypeStruct((CP, C_MID), jnp.float32),
        ],
    )(x_pad, w_cat)


def _tc2_body(x_ref, s1_ref, agg_ref, b1_ref, m_ref):
    i = pl.program_id(0)
    blk = x_ref.shape[0]
    h1 = _lrelu(s1_ref[...] + agg_ref[0, :, :C_MID] + agg_ref[1, :, :C_MID]
                + b1_ref[...])
    row = jax.lax.broadcasted_iota(jnp.int32, (blk, 1), 0) + i * blk
    h1 = jnp.where(row < N_LOW, h1, 0.0)
    m_ref[0] = x_ref[...]
    m_ref[1] = jnp.concatenate([h1, jnp.zeros_like(h1)], axis=1)


def _tc2(x_pad, s1, agg_l, b1):
    blk = CP // 8
    return pl.pallas_call(
        _tc2_body,
        grid=(8,),
        in_specs=[
            pl.BlockSpec((blk, C_IN), lambda i: (i, 0)),
            pl.BlockSpec((blk, C_MID), lambda i: (i, 0)),
            pl.BlockSpec((2, blk, 128), lambda i: (0, i, 0)),
            pl.BlockSpec((1, C_MID), lambda i: (0, 0)),
        ],
        out_specs=pl.BlockSpec((2, blk, 128), lambda i: (0, i, 0)),
        out_shape=jax.ShapeDtypeStruct((2, CP, 128), jnp.float32),
    )(x_pad, s1, agg_l, b1)


def _tc3_body(u_ref, a_ref, wss, ws2, wns, wn2, b2_ref, bsk_ref, o_ref):
    xu, hu = u_ref[0], u_ref[1][:, :C_MID]
    ask, a2 = a_ref[0], a_ref[1][:, :C_MID]
    skip_pre = _dot(xu, wss[...]) + _dot(ask, wns[...]) + bsk_ref[...]
    h2_pre = _dot(hu, ws2[...]) + _dot(a2, wn2[...]) + b2_ref[...]
    o_ref[...] = _lrelu(_lrelu(h2_pre) + _lrelu(skip_pre))


def _tc3(u, agg, w_self_skip, w_self2, w_nbr_skip, w_nbr2, b2, b_skip):
    blk = FP // 16
    wmat = lambda m, n: pl.BlockSpec((m, n), lambda i: (0, 0))
    return pl.pallas_call(
        _tc3_body,
        grid=(16,),
        in_specs=[
            pl.BlockSpec((2, blk, 128), lambda i: (0, i, 0)),
            pl.BlockSpec((2, blk, 128), lambda i: (0, i, 0)),
            wmat(C_IN, C_OUT), wmat(C_MID, C_OUT),
            wmat(C_IN, C_OUT), wmat(C_MID, C_OUT),
            wmat(1, C_OUT), wmat(1, C_OUT),
        ],
        out_specs=pl.BlockSpec((blk, C_OUT), lambda i: (i, 0)),
        out_shape=jax.ShapeDtypeStruct((FP, C_OUT), jnp.float32),
    )(u, agg, w_self_skip, w_self2, w_nbr_skip, w_nbr2,
      b2.reshape(1, C_OUT), b_skip.reshape(1, C_OUT))


_SC_MESH = plsc.VectorSubcoreMesh(core_axis_name="c", subcore_axis_name="s")


def _sc_low_body(y1_hbm, src_hbm, dst_hbm, zeros_hbm, out_hbm,
                 acc_sh, src_v, dst_v, rows_a, rows_b, gsem, ssem):
    c = lax.axis_index("c")
    s = lax.axis_index("s")
    rows_per = CP // 16
    r0 = s * rows_per
    pltpu.sync_copy(zeros_hbm.at[pl.ds(0, 64)], rows_a.at[pl.ds(0, 64)])

    @pl.loop(0, 5)
    def _(k):
        pltpu.sync_copy(rows_a.at[pl.ds(0, 64)],
                        acc_sh.at[pl.ds(r0 + k * 64, 64)])

    plsc.subcore_barrier()
    w = c * 16 + s
    pltpu.sync_copy(src_hbm.at[pl.ds(w * 40, 40)], src_v)
    pltpu.sync_copy(dst_hbm.at[pl.ds(w * 40, 40)], dst_v)

    @pl.loop(0, 20)
    def _(p):
        e = 2 * p
        ga = pltpu.async_copy(y1_hbm.at[src_v.at[e]], rows_a, gsem)
        gb = pltpu.async_copy(y1_hbm.at[src_v.at[e + 1]], rows_b, gsem)
        ga.wait()
        gb.wait()
        sa = pltpu.async_copy(rows_a, acc_sh.at[dst_v.at[e]], ssem, add=True)
        sb = pltpu.async_copy(rows_b, acc_sh.at[dst_v.at[e + 1]], ssem,
                              add=True)
        sa.wait()
        sb.wait()

    plsc.subcore_barrier()

    @pl.loop(0, 5)
    def _(k):
        pltpu.sync_copy(acc_sh.at[pl.ds(r0 + k * 64, 64)],
                        rows_a.at[pl.ds(0, 64)])
        pltpu.sync_copy(rows_a.at[pl.ds(0, 64)],
                        out_hbm.at[c, pl.ds(r0 + k * 64, 64)])


def _sc_low(y1_pad, src_r, dst_r, zeros_l):
    f = pl.kernel(
        _sc_low_body,
        out_type=jax.ShapeDtypeStruct((2, CP, 128), jnp.float32),
        mesh=_SC_MESH,
        scratch_types=[
            pltpu.VMEM_SHARED((CP, 128), jnp.float32),
            pltpu.VMEM((40, 128), jnp.int32),
            pltpu.VMEM((40, 128), jnp.int32),
            pltpu.VMEM((128, 128), jnp.float32),
            pltpu.VMEM((128, 128), jnp.float32),
            pltpu.SemaphoreType.DMA,
            pltpu.SemaphoreType.DMA,
        ],
    )
    return f(y1_pad, src_r, dst_r, zeros_l)


def _sc_high_body(m_hbm, idx_hbm, src_hbm, dst_hbm, zeros_hbm,
                  agg_hbm, u_hbm,
                  acc_sh, sidx_v, src_v, dst_v, rows_a, rows_b, gsem, ssem):
    c = lax.axis_index("c")
    s = lax.axis_index("s")
    rows_per = FP // 16
    r0 = s * rows_per
    pltpu.sync_copy(zeros_hbm.at[pl.ds(0, 128)], rows_a)

    @pl.loop(0, 5)
    def _(k):
        pltpu.sync_copy(rows_a, acc_sh.at[pl.ds(r0 + k * 128, 128)])
        pltpu.sync_copy(rows_a, u_hbm.at[c, pl.ds(r0 + k * 128, 128)])

    plsc.subcore_barrier()
    pltpu.sync_copy(idx_hbm.at[s], sidx_v)

    @pl.loop(0, 5)
    def _(j):
        pltpu.sync_copy(m_hbm.at[c, pl.ds(s * 320 + j * 64, 64)],
                        rows_a.at[pl.ds(0, 64)])
        pltpu.sync_copy(rows_a.at[pl.ds(0, 64)], u_hbm.at[c].at[sidx_v.at[j]])

    plsc.subcore_barrier()
    @pl.loop(0, 10)
    def _(g):
        pltpu.sync_copy(src_hbm.at[pl.ds(s * 160 + g * 16, 16)], src_v)
        pltpu.sync_copy(dst_hbm.at[pl.ds(s * 160 + g * 16, 16)], dst_v)

        @pl.loop(0, 8)
        def _(p):
            e = 2 * p
            ga = pltpu.async_copy(u_hbm.at[c].at[src_v.at[e]], rows_a, gsem)
            gb = pltpu.async_copy(u_hbm.at[c].at[src_v.at[e + 1]], rows_b,
                                  gsem)
            ga.wait()
            gb.wait()
            sa = pltpu.async_copy(rows_a, acc_sh.at[dst_v.at[e]], ssem,
                                  add=True)
            sb = pltpu.async_copy(rows_b, acc_sh.at[dst_v.at[e + 1]], ssem,
                                  add=True)
            sa.wait()
            sb.wait()

    plsc.subcore_barrier()

    @pl.loop(0, 5)
    def _(k):
        pltpu.sync_copy(acc_sh.at[pl.ds(r0 + k * 128, 128)], rows_a)
        pltpu.sync_copy(rows_a, agg_hbm.at[c, pl.ds(r0 + k * 128, 128)])


def _sc_high(m, idx_r, src_r, dst_r, zeros_h):
    f = pl.kernel(
        _sc_high_body,
        out_type=[
            jax.ShapeDtypeStruct((2, FP, 128), jnp.float32),
            jax.ShapeDtypeStruct((2, FP, 128), jnp.float32),
        ],
        mesh=_SC_MESH,
        scratch_types=[
            pltpu.VMEM_SHARED((FP, 128), jnp.float32),
            pltpu.VMEM((8, 64), jnp.int32),
            pltpu.VMEM((16, 128), jnp.int32),
            pltpu.VMEM((16, 128), jnp.int32),
            pltpu.VMEM((128, 128), jnp.float32),
            pltpu.VMEM((128, 128), jnp.float32),
            pltpu.SemaphoreType.DMA,
            pltpu.SemaphoreType.DMA,
        ],
    )
    return f(m, idx_r, src_r, dst_r, zeros_h)


def _pad_edges(e, total, pad_val):
    pad = jnp.full((total - e.shape[0],), pad_val, dtype=jnp.int32)
    return jnp.concatenate([e, pad]).reshape(total // 128, 128)


def kernel(x, edge_index_low, edge_index_high, idx,
           W_self1, W_nbr1, b1, W_self2, W_nbr2, b2,
           W_self_skip, W_nbr_skip, b_skip):
    x_pad = jnp.concatenate(
        [x, jnp.zeros((CP - N_LOW, C_IN), jnp.float32)], axis=0)
    w_cat = jnp.concatenate([W_nbr1, W_self1], axis=1)

    src_l = _pad_edges(edge_index_low[0], ELP, LOW_PAD_NODE)
    dst_l = _pad_edges(edge_index_low[1], ELP, LOW_PAD_NODE)
    src_h = _pad_edges(edge_index_high[0], EHP, HIGH_PAD_NODE)
    dst_h = _pad_edges(edge_index_high[1], EHP, HIGH_PAD_NODE)
    idx_r = jnp.concatenate(
        [idx, jnp.full((CP - N_LOW,), IDX_PAD, jnp.int32)]).reshape(16, 5, 64)
    idx_r = jnp.concatenate(
        [idx_r, jnp.full((16, 3, 64), IDX_PAD, jnp.int32)], axis=1)

    zeros_l = jnp.zeros((CP, 128), jnp.float32)
    zeros_h = jnp.zeros((FP, 128), jnp.float32)

    y1, s1 = _tc1(x_pad, w_cat)
    agg_l = _sc_low(y1, src_l, dst_l, zeros_l)
    m = _tc2(x_pad, s1, agg_l, b1.reshape(1, C_MID))
    agg_h, u = _sc_high(m, idx_r, src_h, dst_h, zeros_h)
    out = _tc3(u, agg_h, W_self_skip, W_self2, W_nbr_skip, W_nbr2,
               b2, b_skip)
    return out[:N_HIGH]


# speedup vs baseline: 3.9126x; 1.0687x over previous
import jax
import jax.numpy as jnp
from jax import lax
from jax.experimental import pallas as pl
from jax.experimental.pallas import tpu as pltpu
from jax.experimental.pallas import tpu_sc as plsc

N_LOW = 5000
N_HIGH = 10000
C_IN = 128
C_MID = 64
C_OUT = 128

CP = 5120
FP = 10240
ELP = 163840
EHP = 327680
LOW_PAD_NODE = 5000
HIGH_PAD_NODE = 10008
IDX_PAD = 10000

_HIGHEST = jax.lax.Precision.HIGHEST


def _lrelu(v):
    return jnp.where(v >= 0, v, 0.01 * v)


def _dot(a, b):
    return jnp.dot(a, b, precision=_HIGHEST, preferred_element_type=jnp.float32)


def _tc1_body(x_ref, w_ref, y1_ref, s1_ref):
    h = _dot(x_ref[...], w_ref[...])
    y1_ref[...] = jnp.concatenate(
        [h[:, :C_MID], jnp.zeros_like(h[:, :C_MID])], axis=1)
    s1_ref[...] = h[:, C_MID:]


def _tc1(x_pad, w_cat):
    return pl.pallas_call(
        _tc1_body,
        grid=(8,),
        in_specs=[
            pl.BlockSpec((CP // 8, C_IN), lambda i: (i, 0)),
            pl.BlockSpec((C_IN, 2 * C_MID), lambda i: (0, 0)),
        ],
        out_specs=[
            pl.BlockSpec((CP // 8, 128), lambda i: (i, 0)),
            pl.BlockSpec((CP // 8, C_MID), lambda i: (i, 0)),
        ],
        out_shape=[
            jax.ShapeDtypeStruct((CP, 128), jnp.float32),
            jax.ShapeDtypeStruct((CP, C_MID), jnp.float32),
        ],
    )(x_pad, w_cat)


def _tc2_body(x_ref, s1_ref, agg_ref, b1_ref, m_ref):
    i = pl.program_id(0)
    blk = x_ref.shape[0]
    h1 = _lrelu(s1_ref[...] + agg_ref[0, :, :C_MID] + agg_ref[1, :, :C_MID]
                + b1_ref[...])
    row = jax.lax.broadcasted_iota(jnp.int32, (blk, 1), 0) + i * blk
    h1 = jnp.where(row < N_LOW, h1, 0.0)
    m_ref[0] = x_ref[...]
    m_ref[1] = jnp.concatenate([h1, jnp.zeros_like(h1)], axis=1)


def _tc2(x_pad, s1, agg_l, b1):
    blk = CP // 8
    return pl.pallas_call(
        _tc2_body,
        grid=(8,),
        in_specs=[
            pl.BlockSpec((blk, C_IN), lambda i: (i, 0)),
            pl.BlockSpec((blk, C_MID), lambda i: (i, 0)),
            pl.BlockSpec((2, blk, 128), lambda i: (0, i, 0)),
            pl.BlockSpec((1, C_MID), lambda i: (0, 0)),
        ],
        out_specs=pl.BlockSpec((2, blk, 128), lambda i: (0, i, 0)),
        out_shape=jax.ShapeDtypeStruct((2, CP, 128), jnp.float32),
    )(x_pad, s1, agg_l, b1)


def _tc3_body(u_ref, a_ref, wss, ws2, wns, wn2, b2_ref, bsk_ref, o_ref):
    xu, hu = u_ref[0], u_ref[1][:, :C_MID]
    ask, a2 = a_ref[0], a_ref[1][:, :C_MID]
    skip_pre = _dot(xu, wss[...]) + _dot(ask, wns[...]) + bsk_ref[...]
    h2_pre = _dot(hu, ws2[...]) + _dot(a2, wn2[...]) + b2_ref[...]
    o_ref[...] = _lrelu(_lrelu(h2_pre) + _lrelu(skip_pre))


def _tc3(u, agg, w_self_skip, w_self2, w_nbr_skip, w_nbr2, b2, b_skip):
    blk = FP // 16
    wmat = lambda m, n: pl.BlockSpec((m, n), lambda i: (0, 0))
    return pl.pallas_call(
        _tc3_body,
        grid=(16,),
        in_specs=[
            pl.BlockSpec((2, blk, 128), lambda i: (0, i, 0)),
            pl.BlockSpec((2, blk, 128), lambda i: (0, i, 0)),
            wmat(C_IN, C_OUT), wmat(C_MID, C_OUT),
            wmat(C_IN, C_OUT), wmat(C_MID, C_OUT),
            wmat(1, C_OUT), wmat(1, C_OUT),
        ],
        out_specs=pl.BlockSpec((blk, C_OUT), lambda i: (i, 0)),
        out_shape=jax.ShapeDtypeStruct((FP, C_OUT), jnp.float32),
    )(u, agg, w_self_skip, w_self2, w_nbr_skip, w_nbr2,
      b2.reshape(1, C_OUT), b_skip.reshape(1, C_OUT))


_SC_MESH = plsc.VectorSubcoreMesh(core_axis_name="c", subcore_axis_name="s")


def _sc_low_body(y1_hbm, src_hbm, dst_hbm, zeros_hbm, out_hbm,
                 acc_sh, src_v, dst_v, rows_a, rows_b, gsem_a, gsem_b,
                 ssem_a, ssem_b):
    c = lax.axis_index("c")
    s = lax.axis_index("s")
    rows_per = CP // 16
    r0 = s * rows_per
    pltpu.sync_copy(zeros_hbm.at[pl.ds(0, 64)], rows_a.at[pl.ds(0, 64)])

    @pl.loop(0, 5)
    def _(k):
        pltpu.sync_copy(rows_a.at[pl.ds(0, 64)],
                        acc_sh.at[pl.ds(r0 + k * 64, 64)])

    plsc.subcore_barrier()
    w = c * 16 + s
    pltpu.sync_copy(src_hbm.at[pl.ds(w * 40, 40)], src_v)
    pltpu.sync_copy(dst_hbm.at[pl.ds(w * 40, 40)], dst_v)

    def _g(e, buf, sem):
        return pltpu.async_copy(y1_hbm.at[src_v.at[e]], buf, sem)

    def _gw(e, buf, sem):
        pltpu.make_async_copy(y1_hbm.at[src_v.at[e]], buf, sem).wait()

    def _s(e, buf, sem):
        return pltpu.async_copy(buf, acc_sh.at[dst_v.at[e]], sem, add=True)

    def _sw(e, buf, sem):
        pltpu.make_async_copy(buf, acc_sh.at[dst_v.at[e]], sem).wait()

    _g(0, rows_a, gsem_a)

    @pl.loop(0, 19)
    def _(q):
        e = 2 * q
        _gw(e, rows_a, gsem_a)
        _g(e + 1, rows_b, gsem_b)
        _s(e, rows_a, ssem_a)
        _gw(e + 1, rows_b, gsem_b)
        _s(e + 1, rows_b, ssem_b)
        _sw(e, rows_a, ssem_a)
        _g(e + 2, rows_a, gsem_a)
        _sw(e + 1, rows_b, ssem_b)

    _gw(38, rows_a, gsem_a)
    _s(38, rows_a, ssem_a)
    _g(39, rows_b, gsem_b)
    _gw(39, rows_b, gsem_b)
    _s(39, rows_b, ssem_b)
    _sw(38, rows_a, ssem_a)
    _sw(39, rows_b, ssem_b)

    plsc.subcore_barrier()

    @pl.loop(0, 5)
    def _(k):
        pltpu.sync_copy(acc_sh.at[pl.ds(r0 + k * 64, 64)],
                        rows_a.at[pl.ds(0, 64)])
        pltpu.sync_copy(rows_a.at[pl.ds(0, 64)],
                        out_hbm.at[c, pl.ds(r0 + k * 64, 64)])


def _sc_low(y1_pad, src_r, dst_r, zeros_l):
    f = pl.kernel(
        _sc_low_body,
        out_type=jax.ShapeDtypeStruct((2, CP, 128), jnp.float32),
        mesh=_SC_MESH,
        scratch_types=[
            pltpu.VMEM_SHARED((CP, 128), jnp.float32),
            pltpu.VMEM((40, 128), jnp.int32),
            pltpu.VMEM((40, 128), jnp.int32),
            pltpu.VMEM((128, 128), jnp.float32),
            pltpu.VMEM((128, 128), jnp.float32),
            pltpu.SemaphoreType.DMA,
            pltpu.SemaphoreType.DMA,
            pltpu.SemaphoreType.DMA,
            pltpu.SemaphoreType.DMA,
        ],
    )
    return f(y1_pad, src_r, dst_r, zeros_l)


def _sc_high_body(m_hbm, idx_hbm, src_hbm, dst_hbm, zeros_hbm,
                  agg_hbm, u_hbm,
                  acc_sh, sidx_v, src_v, dst_v, rows_a, rows_b, gsem_a, gsem_b,
                  ssem_a, ssem_b):
    c = lax.axis_index("c")
    s = lax.axis_index("s")
    rows_per = FP // 16
    r0 = s * rows_per
    pltpu.sync_copy(zeros_hbm.at[pl.ds(0, 128)], rows_a)

    @pl.loop(0, 5)
    def _(k):
        pltpu.sync_copy(rows_a, acc_sh.at[pl.ds(r0 + k * 128, 128)])
        pltpu.sync_copy(rows_a, u_hbm.at[c, pl.ds(r0 + k * 128, 128)])

    plsc.subcore_barrier()
    pltpu.sync_copy(idx_hbm.at[s], sidx_v)

    @pl.loop(0, 5)
    def _(j):
        pltpu.sync_copy(m_hbm.at[c, pl.ds(s * 320 + j * 64, 64)],
                        rows_a.at[pl.ds(0, 64)])
        pltpu.sync_copy(rows_a.at[pl.ds(0, 64)],
                        u_hbm.at[c].at[sidx_v.at[j]])

    plsc.subcore_barrier()
    def _g(e, buf, sem):
        return pltpu.async_copy(u_hbm.at[c].at[src_v.at[e]], buf, sem)

    def _gw(e, buf, sem):
        pltpu.make_async_copy(u_hbm.at[c].at[src_v.at[e]], buf, sem).wait()

    def _s(e, buf, sem):
        return pltpu.async_copy(buf, acc_sh.at[dst_v.at[e]], sem, add=True)

    def _sw(e, buf, sem):
        pltpu.make_async_copy(buf, acc_sh.at[dst_v.at[e]], sem).wait()

    @pl.loop(0, 10)
    def _(g):
        pltpu.sync_copy(src_hbm.at[pl.ds(s * 160 + g * 16, 16)], src_v)
        pltpu.sync_copy(dst_hbm.at[pl.ds(s * 160 + g * 16, 16)], dst_v)
        _g(0, rows_a, gsem_a)

        @pl.loop(0, 7)
        def _(q):
            e = 2 * q
            _gw(e, rows_a, gsem_a)
            _g(e + 1, rows_b, gsem_b)
            _s(e, rows_a, ssem_a)
            _gw(e + 1, rows_b, gsem_b)
            _s(e + 1, rows_b, ssem_b)
            _sw(e, rows_a, ssem_a)
            _g(e + 2, rows_a, gsem_a)
            _sw(e + 1, rows_b, ssem_b)

        _gw(14, rows_a, gsem_a)
        _s(14, rows_a, ssem_a)
        _g(15, rows_b, gsem_b)
        _gw(15, rows_b, gsem_b)
        _s(15, rows_b, ssem_b)
        _sw(14, rows_a, ssem_a)
        _sw(15, rows_b, ssem_b)

    plsc.subcore_barrier()

    @pl.loop(0, 5)
    def _(k):
        pltpu.sync_copy(acc_sh.at[pl.ds(r0 + k * 128, 128)], rows_a)
        pltpu.sync_copy(rows_a, agg_hbm.at[c, pl.ds(r0 + k * 128, 128)])


def _sc_high(m, idx_r, src_r, dst_r, zeros_h):
    f = pl.kernel(
        _sc_high_body,
        out_type=[
            jax.ShapeDtypeStruct((2, FP, 128), jnp.float32),
            jax.ShapeDtypeStruct((2, FP, 128), jnp.float32),
        ],
        mesh=_SC_MESH,
        scratch_types=[
            pltpu.VMEM_SHARED((FP, 128), jnp.float32),
            pltpu.VMEM((8, 64), jnp.int32),
            pltpu.VMEM((16, 128), jnp.int32),
            pltpu.VMEM((16, 128), jnp.int32),
            pltpu.VMEM((128, 128), jnp.float32),
            pltpu.VMEM((128, 128), jnp.float32),
            pltpu.SemaphoreType.DMA,
            pltpu.SemaphoreType.DMA,
            pltpu.SemaphoreType.DMA,
            pltpu.SemaphoreType.DMA,
        ],
    )
    return f(m, idx_r, src_r, dst_r, zeros_h)


def _pad_edges(e, total, pad_val, width=128):
    pad = jnp.full((total - e.shape[0],), pad_val, dtype=jnp.int32)
    return jnp.concatenate([e, pad]).reshape(total // width, width)


def kernel(x, edge_index_low, edge_index_high, idx,
           W_self1, W_nbr1, b1, W_self2, W_nbr2, b2,
           W_self_skip, W_nbr_skip, b_skip):
    x_pad = jnp.concatenate(
        [x, jnp.zeros((CP - N_LOW, C_IN), jnp.float32)], axis=0)
    w_cat = jnp.concatenate([W_nbr1, W_self1], axis=1)

    src_l = _pad_edges(edge_index_low[0], ELP, LOW_PAD_NODE)
    dst_l = _pad_edges(edge_index_low[1], ELP, LOW_PAD_NODE)
    src_h = _pad_edges(edge_index_high[0], EHP, HIGH_PAD_NODE)
    dst_h = _pad_edges(edge_index_high[1], EHP, HIGH_PAD_NODE)
    idx_r = jnp.concatenate(
        [idx, jnp.full((CP - N_LOW,), IDX_PAD, jnp.int32)]).reshape(16, 5, 64)
    idx_r = jnp.concatenate(
        [idx_r, jnp.full((16, 3, 64), IDX_PAD, jnp.int32)], axis=1)

    zeros_l = jnp.zeros((CP, 128), jnp.float32)
    zeros_h = jnp.zeros((FP, 128), jnp.float32)

    y1, s1 = _tc1(x_pad, w_cat)
    agg_l = _sc_low(y1, src_l, dst_l, zeros_l)
    m = _tc2(x_pad, s1, agg_l, b1.reshape(1, C_MID))
    agg_h, u = _sc_high(m, idx_r, src_h, dst_h, zeros_h)
    out = _tc3(u, agg_h, W_self_skip, W_self2, W_nbr_skip, W_nbr2,
               b2, b_skip)
    return out[:N_HIGH]
